# baseline (speedup 1.0000x reference)
import functools

import jax
import jax.numpy as jnp
from jax import lax
from jax.experimental import pallas as pl
from jax.experimental.pallas import tpu as pltpu
from jax.experimental.pallas import tpu_sc as plsc


def _make_gather(B, D, chunk, nbuf):
    info = plsc.get_sparse_core_info()
    NC, NS = info.num_cores, info.num_subcores
    NW = NC * NS
    b_per_w = B // NW
    n_chunks = b_per_w // chunk
    mesh = plsc.VectorSubcoreMesh(core_axis_name="c", subcore_axis_name="s")

    scratch = [pltpu.VMEM((b_per_w,), jnp.int32)]
    scratch += [pltpu.VMEM((chunk, D), jnp.float32) for _ in range(nbuf)]
    scratch += [pltpu.SemaphoreType.DMA for _ in range(2 * nbuf)]

    @functools.partial(
        pl.kernel,
        mesh=mesh,
        out_type=jax.ShapeDtypeStruct((B, D), jnp.float32),
        scratch_types=scratch,
        compiler_params=pltpu.CompilerParams(use_tc_tiling_on_sc=False),
    )
    def gather_kernel(table_hbm, idx_hbm, out_hbm, idx_v, *rest):
        bufs = rest[:nbuf]
        gsems = rest[nbuf : 2 * nbuf]
        wsems = rest[2 * nbuf : 3 * nbuf]
        wid = lax.axis_index("s") * NC + lax.axis_index("c")
        base = wid * b_per_w
        pltpu.sync_copy(idx_hbm.at[pl.ds(base, b_per_w)], idx_v)
        g = [None] * nbuf
        w = [None] * nbuf

        def drain(d):
            s = d % nbuf
            g[s].wait()
            w[s] = pltpu.async_copy(
                bufs[s], out_hbm.at[pl.ds(base + d * chunk, chunk)], wsems[s]
            )

        for c in range(n_chunks):
            s = c % nbuf
            if c >= nbuf:
                w[s].wait()
            g[s] = pltpu.async_copy(
                table_hbm.at[idx_v.at[pl.ds(c * chunk, chunk)]], bufs[s], gsems[s]
            )
            if c >= nbuf - 1:
                drain(c - (nbuf - 1))
        for d in range(max(0, n_chunks - nbuf + 1), n_chunks):
            drain(d)
        for s in range(nbuf):
            if w[s] is not None:
                w[s].wait()

    return gather_kernel


def kernel(input, table):
    seq, batch, _ = input.shape
    vocab, dim = table.shape
    B = seq * batch
    chunk, nbuf = 128, 6
    idx1d = input.reshape(B)
    out = _make_gather(B, dim, chunk, nbuf)(table, idx1d)
    return out.reshape(seq, batch, dim)
